# baseline (speedup 1.0000x reference)
import functools

import jax
import jax.numpy as jnp
from jax import lax
from jax.experimental import pallas as pl
from jax.experimental.pallas import tpu as pltpu
from jax.experimental.pallas import tpu_sc as plsc

N = 10000
E = 160000
D_IN = 500
H = 16
C = 3

N_PAD = 10240
D_PAD = 512
NC = 2
NS = 16
NW = NC * NS
CHUNK = 128
NCHUNK = 40
E_PAD = NW * CHUNK * NCHUNK
EPW = CHUNK * NCHUNK
RPW = N_PAD // NS
BLK = 512
GRID = N_PAD // BLK

@functools.cache
def _sc_kernels():
    mesh = plsc.VectorSubcoreMesh(core_axis_name="c", subcore_axis_name="s")
    params = pltpu.CompilerParams(use_tc_tiling_on_sc=False)

    @functools.partial(
        pl.kernel,
        mesh=mesh,
        compiler_params=params,
        out_type=jax.ShapeDtypeStruct((NC, N_PAD, H), jnp.float32),
        scratch_types=[
            pltpu.VMEM_SHARED((N_PAD, H), jnp.float32),
            pltpu.VMEM((NCHUNK, CHUNK), jnp.int32),
            pltpu.VMEM((CHUNK, H), jnp.float32),
            pltpu.SemaphoreType.DMA,
        ],
    )
    def _sc_degree(ones_hbm, dst_hbm, out_hbm, acc_sh, didx, ones_v, sem_s):
        c = lax.axis_index("c")
        s = lax.axis_index("s")
        w = c * NS + s
        r0 = s * RPW
        pltpu.sync_copy(ones_hbm.at[pl.ds(r0, RPW)], acc_sh.at[pl.ds(r0, RPW)])
        pltpu.sync_copy(ones_hbm.at[pl.ds(0, CHUNK)], ones_v)
        pltpu.sync_copy(dst_hbm.at[pl.ds(w * NCHUNK, NCHUNK)], didx)
        plsc.subcore_barrier()

        def fire(j, carry):
            pltpu.async_copy(ones_v, acc_sh.at[didx.at[j]], sem_s, add=True)
            return carry

        lax.fori_loop(0, NCHUNK, fire, 0)

        def drain(j, carry):
            pltpu.make_async_copy(ones_v, acc_sh.at[didx.at[j]], sem_s).wait()
            return carry

        lax.fori_loop(0, NCHUNK, drain, 0)
        plsc.subcore_barrier()
        pltpu.sync_copy(acc_sh.at[pl.ds(r0, RPW)], out_hbm.at[c, pl.ds(r0, RPW)])

    RPT = N_PAD // NW

    @functools.partial(
        pl.kernel,
        mesh=mesh,
        compiler_params=params,
        out_type=jax.ShapeDtypeStruct((N_PAD, H), jnp.float32),
        scratch_types=[
            pltpu.VMEM((RPT, 128), jnp.float32),
            pltpu.VMEM((RPT, H), jnp.float32),
        ],
    )
    def _sc_repack(ypad_hbm, out_hbm, stage_a, stage_b):
        c = lax.axis_index("c")
        s = lax.axis_index("s")
        w = c * NS + s
        r0 = w * RPT
        pltpu.sync_copy(ypad_hbm.at[pl.ds(r0, RPT)], stage_a)

        def repack(r, carry):
            stage_b[r, :] = stage_a[r, pl.ds(0, H)]
            return carry

        lax.fori_loop(0, RPT, repack, 0)
        pltpu.sync_copy(stage_b, out_hbm.at[pl.ds(r0, RPT)])

    @functools.partial(
        pl.kernel,
        mesh=mesh,
        compiler_params=params,
        out_type=jax.ShapeDtypeStruct((NC, N_PAD, H), jnp.float32),
        scratch_types=[
            pltpu.VMEM_SHARED((N_PAD, H), jnp.float32),
            pltpu.VMEM((NCHUNK, CHUNK), jnp.int32),
            pltpu.VMEM((NCHUNK, CHUNK), jnp.int32),
            pltpu.VMEM((NCHUNK, CHUNK, H), jnp.float32),
            pltpu.SemaphoreType.DMA,
            pltpu.SemaphoreType.DMA,
        ],
    )
    def _sc_propagate(y_hbm, src_hbm, dst_hbm, out_hbm, acc_sh, sidx,
                      didx, rows, sem_g, sem_s):
        c = lax.axis_index("c")
        s = lax.axis_index("s")
        w = c * NS + s
        r0 = s * RPW
        pltpu.sync_copy(y_hbm.at[pl.ds(r0, RPW)], acc_sh.at[pl.ds(r0, RPW)])
        pltpu.sync_copy(src_hbm.at[pl.ds(w * NCHUNK, NCHUNK)], sidx)
        pltpu.sync_copy(dst_hbm.at[pl.ds(w * NCHUNK, NCHUNK)], didx)
        plsc.subcore_barrier()

        def fire_g(j, carry):
            pltpu.async_copy(y_hbm.at[sidx.at[j]], rows.at[j], sem_g)
            return carry

        lax.fori_loop(0, NCHUNK, fire_g, 0)

        def drain_g(j, carry):
            pltpu.make_async_copy(y_hbm.at[sidx.at[j]], rows.at[j],
                                  sem_g).wait()
            return carry

        lax.fori_loop(0, NCHUNK, drain_g, 0)

        def fire_s(j, carry):
            pltpu.async_copy(rows.at[j], acc_sh.at[didx.at[j]], sem_s,
                             add=True)
            return carry

        lax.fori_loop(0, NCHUNK, fire_s, 0)

        def drain_s(j, carry):
            pltpu.make_async_copy(rows.at[j], acc_sh.at[didx.at[j]],
                                  sem_s).wait()
            return carry

        lax.fori_loop(0, NCHUNK, drain_s, 0)
        plsc.subcore_barrier()
        pltpu.sync_copy(acc_sh.at[pl.ds(r0, RPW)], out_hbm.at[c, pl.ds(r0, RPW)])

    return _sc_degree, _sc_repack, _sc_propagate


def _dinv(degp0, degp1):
    deg = degp0 + degp1 - 1.0
    return lax.rsqrt(jnp.maximum(deg, 1.0))


PBLK = BLK * H // 128
PN = N_PAD * H // 128


def _pad_lanes(a):
    return jnp.concatenate([a, jnp.zeros((BLK, 128 - H), jnp.float32)], axis=1)


def _mm_scale_body(x_ref, w_ref, degp_ref, y_ref):
    xw = jnp.dot(x_ref[...], w_ref[...], preferred_element_type=jnp.float32)
    y_ref[...] = _pad_lanes(xw * _dinv(degp_ref[0], degp_ref[1]))


def _mid_body(accp_ref, degp_ref, y1_ref, b1_ref, yh_ref):
    i = pl.program_id(0)
    dinv = _dinv(degp_ref[0], degp_ref[1])
    acc = accp_ref[0] + accp_ref[1] - y1_ref[:, :H]
    h = jnp.maximum(dinv * acc + b1_ref[...], 0.0)
    rid = i * BLK + lax.broadcasted_iota(jnp.int32, (BLK, H), 0)
    yh_ref[...] = _pad_lanes(jnp.where(rid < N, dinv * h, 0.0))


def _final_body(accp_ref, degp_ref, yh_ref, w2t_ref, b2_ref, out_ref):
    dinv = _dinv(degp_ref[0], degp_ref[1])
    z = dinv * (accp_ref[0] + accp_ref[1] - yh_ref[:, :H])
    w2t = w2t_ref[...]
    b2 = b2_ref[...]
    ls = [
        jnp.sum(z * w2t[c_:c_ + 1, :], axis=1, keepdims=True) + b2[0, c_]
        for c_ in range(C)
    ]
    m = jnp.maximum(jnp.maximum(ls[0], ls[1]), ls[2])
    se = sum(jnp.exp(l - m) for l in ls)
    lse = m + jnp.log(se)
    out_ref[...] = jnp.concatenate([l - lse for l in ls], axis=1)


_degp_spec = pl.BlockSpec((NC, BLK, H), lambda i: (0, i, 0))
_packed_spec = pl.BlockSpec((BLK, 128), lambda i: (i, 0))
_packed_shape = jax.ShapeDtypeStruct((N_PAD, 128), jnp.float32)

_mm_scale = pl.pallas_call(
    _mm_scale_body,
    grid=(GRID,),
    in_specs=[
        pl.BlockSpec((BLK, D_PAD), lambda i: (i, 0)),
        pl.BlockSpec((D_PAD, H), lambda i: (0, 0)),
        _degp_spec,
    ],
    out_specs=_packed_spec,
    out_shape=_packed_shape,
)

_mid = pl.pallas_call(
    _mid_body,
    grid=(GRID,),
    in_specs=[
        _degp_spec,
        _degp_spec,
        _packed_spec,
        pl.BlockSpec((1, H), lambda i: (0, 0)),
    ],
    out_specs=_packed_spec,
    out_shape=_packed_shape,
)

_final = pl.pallas_call(
    _final_body,
    grid=(GRID,),
    in_specs=[
        _degp_spec,
        _degp_spec,
        _packed_spec,
        pl.BlockSpec((C, H), lambda i: (0, 0)),
        pl.BlockSpec((1, C), lambda i: (0, 0)),
    ],
    out_specs=pl.BlockSpec((BLK, C), lambda i: (i, 0)),
    out_shape=jax.ShapeDtypeStruct((N_PAD, C), jnp.float32),
)


def kernel(x, edge_index, W1, b1, W2, b2):
    src = edge_index[0]
    dst = edge_index[1]
    npad = E_PAD - E
    pad_ids = (N + (jnp.arange(npad, dtype=jnp.int32) % (N_PAD - N)))
    pad_ids = pad_ids.astype(jnp.int32)
    src_p = jnp.concatenate([src, pad_ids]).reshape(E_PAD // CHUNK, CHUNK)
    dst_p = jnp.concatenate([dst, pad_ids]).reshape(E_PAD // CHUNK, CHUNK)

    xp = jnp.zeros((N_PAD, D_PAD), jnp.float32).at[:N, :D_IN].set(x)
    w1p = jnp.zeros((D_PAD, H), jnp.float32).at[:D_IN].set(W1)
    ones = jnp.ones((N_PAD, H), jnp.float32)

    sc_degree, sc_repack, sc_propagate = _sc_kernels()
    degp = sc_degree(ones, dst_p)
    y1 = _mm_scale(xp, w1p, degp)
    acc1 = sc_propagate(sc_repack(y1), src_p, dst_p)
    yh = _mid(acc1, degp, y1, b1.reshape(1, H))
    acc2 = sc_propagate(sc_repack(yh), src_p, dst_p)
    out = _final(acc2, degp, yh, W2.T, b2.reshape(1, C))
    return out[:N]


# speedup vs baseline: 1.1063x; 1.1063x over previous
import functools

import jax
import jax.numpy as jnp
from jax import lax
from jax.experimental import pallas as pl
from jax.experimental.pallas import tpu as pltpu
from jax.experimental.pallas import tpu_sc as plsc

N = 10000
E = 160000
D_IN = 500
H = 16
C = 3

N_PAD = 10240
D_PAD = 512
NC = 2
NS = 16
NW = NC * NS
CHUNK = 128
NCHUNK = 40
E_PAD = NW * CHUNK * NCHUNK
EPW = CHUNK * NCHUNK
RPW = N_PAD // NS
RPT = N_PAD // NW
BLK = 512
GRID = N_PAD // BLK
PBLK = BLK * H // 128
PN = N_PAD * H // 128
PMASK = N * H // 128


@functools.cache
def _sc_kernels():
    mesh = plsc.VectorSubcoreMesh(core_axis_name="c", subcore_axis_name="s")
    params = pltpu.CompilerParams(use_tc_tiling_on_sc=False)
    params_nl = pltpu.CompilerParams(use_tc_tiling_on_sc=False,
                                     needs_layout_passes=False)

    @functools.partial(
        pl.kernel,
        mesh=mesh,
        compiler_params=params,
        out_type=jax.ShapeDtypeStruct((NC, N_PAD, H), jnp.float32),
        scratch_types=[
            pltpu.VMEM_SHARED((N_PAD, H), jnp.float32),
            pltpu.VMEM((NCHUNK, CHUNK), jnp.int32),
            pltpu.VMEM((CHUNK, H), jnp.float32),
            pltpu.SemaphoreType.DMA,
        ],
    )
    def _sc_degree(ones_hbm, dst_hbm, out_hbm, acc_sh, didx, ones_v, sem_s):
        c = lax.axis_index("c")
        s = lax.axis_index("s")
        w = c * NS + s
        r0 = s * RPW
        pltpu.sync_copy(ones_hbm.at[pl.ds(r0, RPW)], acc_sh.at[pl.ds(r0, RPW)])
        pltpu.sync_copy(ones_hbm.at[pl.ds(0, CHUNK)], ones_v)
        pltpu.sync_copy(dst_hbm.at[pl.ds(w * NCHUNK, NCHUNK)], didx)
        plsc.subcore_barrier()

        def fire(j, carry):
            pltpu.async_copy(ones_v, acc_sh.at[didx.at[j]], sem_s, add=True)
            return carry

        lax.fori_loop(0, NCHUNK, fire, 0)

        def drain(j, carry):
            pltpu.make_async_copy(ones_v, acc_sh.at[didx.at[j]], sem_s).wait()
            return carry

        lax.fori_loop(0, NCHUNK, drain, 0)
        plsc.subcore_barrier()
        pltpu.sync_copy(acc_sh.at[pl.ds(r0, RPW)], out_hbm.at[c, pl.ds(r0, RPW)])

    @functools.partial(
        pl.kernel,
        mesh=mesh,
        compiler_params=params_nl,
        out_type=jax.ShapeDtypeStruct((N_PAD, H), jnp.float32),
        scratch_types=[
            pltpu.VMEM((RPT, H), jnp.float32),
            pltpu.VMEM((RPT, H), jnp.float32),
            pltpu.VMEM((RPT, H), jnp.float32),
            pltpu.VMEM((RPT, H), jnp.float32),
        ],
    )
    def _sc_scale(xw_hbm, degp_hbm, y_hbm, xw_v, d0_v, d1_v, y_v):
        c = lax.axis_index("c")
        s = lax.axis_index("s")
        w = c * NS + s
        r0 = w * RPT
        pltpu.sync_copy(xw_hbm.at[pl.ds(r0, RPT)], xw_v)
        pltpu.sync_copy(degp_hbm.at[0, pl.ds(r0, RPT)], d0_v)
        pltpu.sync_copy(degp_hbm.at[1, pl.ds(r0, RPT)], d1_v)

        def row(r, carry):
            d = d0_v[r, :] + d1_v[r, :] - 1.0
            seed = jnp.int32(0x5F3759DF) - lax.shift_right_arithmetic(
                plsc.bitcast(d, jnp.int32), 1)
            v = plsc.bitcast(seed, jnp.float32)
            for _ in range(3):
                v = v * (1.5 - 0.5 * d * v * v)
            y_v[r, :] = xw_v[r, :] * v
            return carry

        lax.fori_loop(0, RPT, row, 0)
        pltpu.sync_copy(y_v, y_hbm.at[pl.ds(r0, RPT)])

    @functools.partial(
        pl.kernel,
        mesh=mesh,
        compiler_params=params,
        out_type=jax.ShapeDtypeStruct((NC, N_PAD, H), jnp.float32),
        scratch_types=[
            pltpu.VMEM_SHARED((N_PAD, H), jnp.float32),
            pltpu.VMEM((NCHUNK, CHUNK), jnp.int32),
            pltpu.VMEM((NCHUNK, CHUNK), jnp.int32),
            pltpu.VMEM((NCHUNK, CHUNK, H), jnp.float32),
            pltpu.SemaphoreType.DMA,
            pltpu.SemaphoreType.DMA,
        ],
    )
    def _sc_propagate(y_hbm, src_hbm, dst_hbm, out_hbm, acc_sh, sidx,
                      didx, rows, sem_g, sem_s):
        c = lax.axis_index("c")
        s = lax.axis_index("s")
        w = c * NS + s
        r0 = s * RPW
        pltpu.sync_copy(y_hbm.at[pl.ds(r0, RPW)], acc_sh.at[pl.ds(r0, RPW)])
        pltpu.sync_copy(src_hbm.at[pl.ds(w * NCHUNK, NCHUNK)], sidx)
        pltpu.sync_copy(dst_hbm.at[pl.ds(w * NCHUNK, NCHUNK)], didx)
        plsc.subcore_barrier()

        def fire_g(j, carry):
            pltpu.async_copy(y_hbm.at[sidx.at[j]], rows.at[j], sem_g)
            return carry

        lax.fori_loop(0, NCHUNK, fire_g, 0)

        def drain_g(j, carry):
            pltpu.make_async_copy(y_hbm.at[sidx.at[j]], rows.at[j],
                                  sem_g).wait()
            return carry

        lax.fori_loop(0, NCHUNK, drain_g, 0)

        def fire_s(j, carry):
            pltpu.async_copy(rows.at[j], acc_sh.at[didx.at[j]], sem_s,
                             add=True)
            return carry

        lax.fori_loop(0, NCHUNK, fire_s, 0)

        def drain_s(j, carry):
            pltpu.make_async_copy(rows.at[j], acc_sh.at[didx.at[j]],
                                  sem_s).wait()
            return carry

        lax.fori_loop(0, NCHUNK, drain_s, 0)
        plsc.subcore_barrier()
        pltpu.sync_copy(acc_sh.at[pl.ds(r0, RPW)], out_hbm.at[c, pl.ds(r0, RPW)])

    return _sc_degree, _sc_scale, _sc_propagate


def _pdinv(degp0, degp1):
    return lax.rsqrt(jnp.maximum(degp0 + degp1 - 1.0, 1.0))


def _mm_body(x_ref, w_ref, xw_ref):
    xw_ref[...] = jnp.dot(x_ref[...], w_ref[...],
                          preferred_element_type=jnp.float32)


def _mid_body(accp_ref, degp_ref, y1_ref, b1t_ref, yh_ref):
    i = pl.program_id(0)
    dinv = _pdinv(degp_ref[0], degp_ref[1])
    acc = accp_ref[0] + accp_ref[1] - y1_ref[...]
    h = jnp.maximum(dinv * acc + b1t_ref[...], 0.0)
    pr = i * PBLK + lax.broadcasted_iota(jnp.int32, (PBLK, 128), 0)
    yh_ref[...] = jnp.where(pr < PMASK, dinv * h, 0.0)


def _t_body(accp_ref, degp_ref, yh_ref, t_ref):
    dinv = _pdinv(degp_ref[0], degp_ref[1])
    t_ref[...] = dinv * (accp_ref[0] + accp_ref[1] - yh_ref[...])


def _final_body(t_ref, w2t_ref, b2_ref, out_ref):
    z = t_ref[...]
    w2t = w2t_ref[...]
    b2 = b2_ref[...]
    ls = [
        jnp.sum(z * w2t[c_:c_ + 1, :], axis=1, keepdims=True) + b2[0, c_]
        for c_ in range(C)
    ]
    m = jnp.maximum(jnp.maximum(ls[0], ls[1]), ls[2])
    se = sum(jnp.exp(l - m) for l in ls)
    lse = m + jnp.log(se)
    out_ref[...] = jnp.concatenate([l - lse for l in ls], axis=1)


_pk_spec = pl.BlockSpec((PBLK, 128), lambda i: (i, 0))
_pk_shape = jax.ShapeDtypeStruct((PN, 128), jnp.float32)
_pk2_spec = pl.BlockSpec((NC, PBLK, 128), lambda i: (0, i, 0))

_mm = pl.pallas_call(
    _mm_body,
    grid=(GRID,),
    in_specs=[
        pl.BlockSpec((BLK, D_PAD), lambda i: (i, 0)),
        pl.BlockSpec((D_PAD, H), lambda i: (0, 0)),
    ],
    out_specs=pl.BlockSpec((BLK, H), lambda i: (i, 0)),
    out_shape=jax.ShapeDtypeStruct((N_PAD, H), jnp.float32),
)

_mid = pl.pallas_call(
    _mid_body,
    grid=(GRID,),
    in_specs=[
        _pk2_spec,
        _pk2_spec,
        _pk_spec,
        pl.BlockSpec((1, 128), lambda i: (0, 0)),
    ],
    out_specs=_pk_spec,
    out_shape=_pk_shape,
)

_t = pl.pallas_call(
    _t_body,
    grid=(GRID,),
    in_specs=[_pk2_spec, _pk2_spec, _pk_spec],
    out_specs=_pk_spec,
    out_shape=_pk_shape,
)

_final = pl.pallas_call(
    _final_body,
    grid=(GRID,),
    in_specs=[
        pl.BlockSpec((BLK, H), lambda i: (i, 0)),
        pl.BlockSpec((C, H), lambda i: (0, 0)),
        pl.BlockSpec((1, C), lambda i: (0, 0)),
    ],
    out_specs=pl.BlockSpec((BLK, C), lambda i: (i, 0)),
    out_shape=jax.ShapeDtypeStruct((N_PAD, C), jnp.float32),
)


def kernel(x, edge_index, W1, b1, W2, b2):
    src = edge_index[0]
    dst = edge_index[1]
    npad = E_PAD - E
    pad_ids = (N + (jnp.arange(npad, dtype=jnp.int32) % (N_PAD - N)))
    pad_ids = pad_ids.astype(jnp.int32)
    src_p = jnp.concatenate([src, pad_ids]).reshape(E_PAD // CHUNK, CHUNK)
    dst_p = jnp.concatenate([dst, pad_ids]).reshape(E_PAD // CHUNK, CHUNK)

    xp = jnp.zeros((N_PAD, D_PAD), jnp.float32).at[:N, :D_IN].set(x)
    w1p = jnp.zeros((D_PAD, H), jnp.float32).at[:D_IN].set(W1)
    ones = jnp.ones((N_PAD, H), jnp.float32)
    b1t = jnp.tile(b1, 128 // H).reshape(1, 128)

    sc_degree, sc_scale, sc_propagate = _sc_kernels()
    degp = sc_degree(ones, dst_p)
    xw = _mm(xp, w1p)
    y1 = sc_scale(xw, degp)
    acc1 = sc_propagate(y1, src_p, dst_p)
    degpp = degp.reshape(NC, PN, 128)
    yhp = _mid(acc1.reshape(NC, PN, 128), degpp, y1.reshape(PN, 128), b1t)
    acc2 = sc_propagate(yhp.reshape(N_PAD, H), src_p, dst_p)
    tp = _t(acc2.reshape(NC, PN, 128), degpp, yhp)
    out = _final(tp.reshape(N_PAD, H), W2.T, b2.reshape(1, C))
    return out[:N]


# speedup vs baseline: 1.8399x; 1.6632x over previous
import functools

import jax
import jax.numpy as jnp
from jax import lax
from jax.experimental import pallas as pl
from jax.experimental.pallas import tpu as pltpu
from jax.experimental.pallas import tpu_sc as plsc

N = 10000
E = 160000
D_IN = 500
H = 16
C = 3

N_PAD = 10240
D_PAD = 512
NC = 2
NS = 16
NW = NC * NS
CHUNK = 128
NCHUNK = 40
E_PAD = NW * CHUNK * NCHUNK
EPW = CHUNK * NCHUNK
RPW = N_PAD // NS
RPT = N_PAD // NW
BLK = 512
GRID = N_PAD // BLK
PBLK = BLK * H // 128
PN = N_PAD * H // 128
PMASK = N * H // 128


@functools.cache
def _sc_kernels():
    mesh = plsc.VectorSubcoreMesh(core_axis_name="c", subcore_axis_name="s")
    params = pltpu.CompilerParams(use_tc_tiling_on_sc=False)
    params_nl = pltpu.CompilerParams(use_tc_tiling_on_sc=False,
                                     needs_layout_passes=False)

    @functools.partial(
        pl.kernel,
        mesh=mesh,
        compiler_params=params,
        out_type=jax.ShapeDtypeStruct((NC, N_PAD, H), jnp.float32),
        scratch_types=[
            pltpu.VMEM_SHARED((N_PAD, H), jnp.float32),
            pltpu.VMEM((NCHUNK, CHUNK), jnp.int32),
            pltpu.VMEM((CHUNK, H), jnp.float32),
            pltpu.SemaphoreType.DMA,
        ],
    )
    def _sc_degree(ones_hbm, dst_hbm, out_hbm, acc_sh, didx, ones_v, sem_s):
        c = lax.axis_index("c")
        s = lax.axis_index("s")
        w = c * NS + s
        r0 = s * RPW
        pltpu.sync_copy(ones_hbm.at[pl.ds(r0, RPW)], acc_sh.at[pl.ds(r0, RPW)])
        pltpu.sync_copy(ones_hbm.at[pl.ds(0, CHUNK)], ones_v)
        pltpu.sync_copy(dst_hbm.at[pl.ds(w * NCHUNK, NCHUNK)], didx)
        plsc.subcore_barrier()

        def fire(j, carry):
            pltpu.async_copy(ones_v, acc_sh.at[didx.at[j]], sem_s, add=True)
            return carry

        lax.fori_loop(0, NCHUNK, fire, 0)

        def drain(j, carry):
            pltpu.make_async_copy(ones_v, acc_sh.at[didx.at[j]], sem_s).wait()
            return carry

        lax.fori_loop(0, NCHUNK, drain, 0)
        plsc.subcore_barrier()
        pltpu.sync_copy(acc_sh.at[pl.ds(r0, RPW)], out_hbm.at[c, pl.ds(r0, RPW)])

    @functools.partial(
        pl.kernel,
        mesh=mesh,
        compiler_params=params_nl,
        out_type=jax.ShapeDtypeStruct((N_PAD, H), jnp.float32),
        scratch_types=[
            pltpu.VMEM((RPT, H), jnp.float32),
            pltpu.VMEM((RPT, H), jnp.float32),
            pltpu.VMEM((RPT, H), jnp.float32),
            pltpu.VMEM((RPT, H), jnp.float32),
        ],
    )
    def _sc_scale(xw_hbm, degp_hbm, y_hbm, xw_v, d0_v, d1_v, y_v):
        c = lax.axis_index("c")
        s = lax.axis_index("s")
        w = c * NS + s
        r0 = w * RPT
        pltpu.sync_copy(xw_hbm.at[pl.ds(r0, RPT)], xw_v)
        pltpu.sync_copy(degp_hbm.at[0, pl.ds(r0, RPT)], d0_v)
        pltpu.sync_copy(degp_hbm.at[1, pl.ds(r0, RPT)], d1_v)

        def row(r, carry):
            d = d0_v[r, :] + d1_v[r, :] - 1.0
            seed = jnp.int32(0x5F3759DF) - lax.shift_right_arithmetic(
                plsc.bitcast(d, jnp.int32), 1)
            v = plsc.bitcast(seed, jnp.float32)
            for _ in range(3):
                v = v * (1.5 - 0.5 * d * v * v)
            y_v[r, :] = xw_v[r, :] * v
            return carry

        lax.fori_loop(0, RPT, row, 0)
        pltpu.sync_copy(y_v, y_hbm.at[pl.ds(r0, RPT)])

    @functools.partial(
        pl.kernel,
        mesh=mesh,
        compiler_params=params,
        out_type=jax.ShapeDtypeStruct((NC, N_PAD, H), jnp.float32),
        scratch_types=[
            pltpu.VMEM_SHARED((N_PAD, H), jnp.float32),
            pltpu.VMEM((NCHUNK, CHUNK), jnp.int32),
            pltpu.VMEM((NCHUNK, CHUNK), jnp.int32),
            pltpu.VMEM((NCHUNK, CHUNK, H), jnp.float32),
            pltpu.SemaphoreType.DMA,
            pltpu.SemaphoreType.DMA,
        ],
    )
    def _sc_propagate(y_hbm, src_hbm, dst_hbm, out_hbm, acc_sh, sidx,
                      didx, rows, sem_g, sem_s):
        c = lax.axis_index("c")
        s = lax.axis_index("s")
        w = c * NS + s
        r0 = s * RPW
        pltpu.sync_copy(y_hbm.at[pl.ds(r0, RPW)], acc_sh.at[pl.ds(r0, RPW)])
        pltpu.sync_copy(src_hbm.at[pl.ds(w * NCHUNK, NCHUNK)], sidx)
        pltpu.sync_copy(dst_hbm.at[pl.ds(w * NCHUNK, NCHUNK)], didx)
        plsc.subcore_barrier()

        def fire_g(j, carry):
            pltpu.async_copy(y_hbm.at[sidx.at[j]], rows.at[j], sem_g)
            return carry

        lax.fori_loop(0, NCHUNK, fire_g, 0)

        def drain_g(j, carry):
            pltpu.make_async_copy(y_hbm.at[sidx.at[j]], rows.at[j],
                                  sem_g).wait()
            return carry

        lax.fori_loop(0, NCHUNK, drain_g, 0)

        def fire_s(j, carry):
            pltpu.async_copy(rows.at[j], acc_sh.at[didx.at[j]], sem_s,
                             add=True)
            return carry

        lax.fori_loop(0, NCHUNK, fire_s, 0)

        def drain_s(j, carry):
            pltpu.make_async_copy(rows.at[j], acc_sh.at[didx.at[j]],
                                  sem_s).wait()
            return carry

        lax.fori_loop(0, NCHUNK, drain_s, 0)
        plsc.subcore_barrier()
        pltpu.sync_copy(acc_sh.at[pl.ds(r0, RPW)], out_hbm.at[c, pl.ds(r0, RPW)])

    return _sc_degree, _sc_scale, _sc_propagate


def _pdinv(degp0, degp1):
    return lax.rsqrt(jnp.maximum(degp0 + degp1 - 1.0, 1.0))


def _mm_body(xt_ref, w_ref, xw_ref):
    i = pl.program_id(0)
    xw = lax.dot_general(xt_ref[...], w_ref[...], (((0,), (0,)), ((), ())),
                         preferred_element_type=jnp.float32)
    rid = i * BLK + lax.broadcasted_iota(jnp.int32, (BLK, H), 0)
    xw_ref[...] = jnp.where(rid < N, xw, 0.0)


def _mid_body(accp_ref, degp_ref, y1_ref, b1t_ref, yh_ref):
    i = pl.program_id(0)
    dinv = _pdinv(degp_ref[0], degp_ref[1])
    acc = accp_ref[0] + accp_ref[1] - y1_ref[...]
    h = jnp.maximum(dinv * acc + b1t_ref[...], 0.0)
    pr = i * PBLK + lax.broadcasted_iota(jnp.int32, (PBLK, 128), 0)
    yh_ref[...] = jnp.where(pr < PMASK, dinv * h, 0.0)


def _t_body(accp_ref, degp_ref, yh_ref, t_ref):
    dinv = _pdinv(degp_ref[0], degp_ref[1])
    t_ref[...] = dinv * (accp_ref[0] + accp_ref[1] - yh_ref[...])


def _final_body(t_ref, w2t_ref, b2_ref, out_ref):
    z = t_ref[...]
    w2t = w2t_ref[...]
    b2 = b2_ref[...]
    ls = [
        jnp.sum(z * w2t[c_:c_ + 1, :], axis=1, keepdims=True) + b2[0, c_]
        for c_ in range(C)
    ]
    m = jnp.maximum(jnp.maximum(ls[0], ls[1]), ls[2])
    se = sum(jnp.exp(l - m) for l in ls)
    lse = m + jnp.log(se)
    out_ref[...] = jnp.concatenate([l - lse for l in ls], axis=1)


_pk_spec = pl.BlockSpec((PBLK, 128), lambda i: (i, 0))
_pk_shape = jax.ShapeDtypeStruct((PN, 128), jnp.float32)
_pk2_spec = pl.BlockSpec((NC, PBLK, 128), lambda i: (0, i, 0))

_mm = pl.pallas_call(
    _mm_body,
    grid=(GRID,),
    in_specs=[
        pl.BlockSpec((D_IN, BLK), lambda i: (0, i)),
        pl.BlockSpec((D_IN, H), lambda i: (0, 0)),
    ],
    out_specs=pl.BlockSpec((BLK, H), lambda i: (i, 0)),
    out_shape=jax.ShapeDtypeStruct((N_PAD, H), jnp.float32),
)

_mid = pl.pallas_call(
    _mid_body,
    grid=(GRID,),
    in_specs=[
        _pk2_spec,
        _pk2_spec,
        _pk_spec,
        pl.BlockSpec((1, 128), lambda i: (0, 0)),
    ],
    out_specs=_pk_spec,
    out_shape=_pk_shape,
)

_t = pl.pallas_call(
    _t_body,
    grid=(GRID,),
    in_specs=[_pk2_spec, _pk2_spec, _pk_spec],
    out_specs=_pk_spec,
    out_shape=_pk_shape,
)

_final = pl.pallas_call(
    _final_body,
    grid=(GRID,),
    in_specs=[
        pl.BlockSpec((BLK, H), lambda i: (i, 0)),
        pl.BlockSpec((C, H), lambda i: (0, 0)),
        pl.BlockSpec((1, C), lambda i: (0, 0)),
    ],
    out_specs=pl.BlockSpec((BLK, C), lambda i: (i, 0)),
    out_shape=jax.ShapeDtypeStruct((N_PAD, C), jnp.float32),
)


def kernel(x, edge_index, W1, b1, W2, b2):
    src = edge_index[0]
    dst = edge_index[1]
    npad = E_PAD - E
    pad_ids = (N + (jnp.arange(npad, dtype=jnp.int32) % (N_PAD - N)))
    pad_ids = pad_ids.astype(jnp.int32)
    src_p = jnp.concatenate([src, pad_ids]).reshape(E_PAD // CHUNK, CHUNK)
    dst_p = jnp.concatenate([dst, pad_ids]).reshape(E_PAD // CHUNK, CHUNK)

    ones = jnp.ones((N_PAD, H), jnp.float32)
    b1t = jnp.tile(b1, 128 // H).reshape(1, 128)

    sc_degree, sc_scale, sc_propagate = _sc_kernels()
    degp = sc_degree(ones, dst_p)
    xw = _mm(x.T, W1)
    y1 = sc_scale(xw, degp)
    acc1 = sc_propagate(y1, src_p, dst_p)
    degpp = degp.reshape(NC, PN, 128)
    yhp = _mid(acc1.reshape(NC, PN, 128), degpp, y1.reshape(PN, 128), b1t)
    acc2 = sc_propagate(yhp.reshape(N_PAD, H), src_p, dst_p)
    tp = _t(acc2.reshape(NC, PN, 128), degpp, yhp)
    out = _final(tp.reshape(N_PAD, H), W2.T, b2.reshape(1, C))
    return out[:N]


# speedup vs baseline: 1.8929x; 1.0288x over previous
import functools

import jax
import jax.numpy as jnp
from jax import lax
from jax.experimental import pallas as pl
from jax.experimental.pallas import tpu as pltpu
from jax.experimental.pallas import tpu_sc as plsc

N = 10000
E = 160000
D_IN = 500
H = 16
C = 3

N_PAD = 10240
D_PAD = 512
NC = 2
NS = 16
NW = NC * NS
CHUNK = 128
NCHUNK = 40
E_PAD = NW * CHUNK * NCHUNK
EPW = CHUNK * NCHUNK
RPW = N_PAD // NS
RPT = N_PAD // NW
BLK = 512
GRID = N_PAD // BLK
PBLK = BLK * H // 128
PN = N_PAD * H // 128
PMASK = N * H // 128


@functools.cache
def _sc_kernels():
    mesh = plsc.VectorSubcoreMesh(core_axis_name="c", subcore_axis_name="s")
    params = pltpu.CompilerParams(use_tc_tiling_on_sc=False)
    params_nl = pltpu.CompilerParams(use_tc_tiling_on_sc=False,
                                     needs_layout_passes=False)

    @functools.partial(
        pl.kernel,
        mesh=mesh,
        compiler_params=params,
        out_type=jax.ShapeDtypeStruct((NC, N_PAD, H), jnp.float32),
        scratch_types=[
            pltpu.VMEM_SHARED((N_PAD, H), jnp.float32),
            pltpu.VMEM((NCHUNK, CHUNK), jnp.int32),
            pltpu.VMEM((CHUNK, H), jnp.float32),
            pltpu.SemaphoreType.DMA,
        ],
    )
    def _sc_degree(ones_hbm, dst_hbm, out_hbm, acc_sh, didx, ones_v, sem_s):
        c = lax.axis_index("c")
        s = lax.axis_index("s")
        w = c * NS + s
        r0 = s * RPW
        pltpu.sync_copy(ones_hbm.at[pl.ds(r0, RPW)], acc_sh.at[pl.ds(r0, RPW)])
        pltpu.sync_copy(ones_hbm.at[pl.ds(0, CHUNK)], ones_v)
        pltpu.sync_copy(dst_hbm.at[pl.ds(w * NCHUNK, NCHUNK)], didx)
        plsc.subcore_barrier()

        def fire(j, carry):
            pltpu.async_copy(ones_v, acc_sh.at[didx.at[j]], sem_s, add=True)
            return carry

        lax.fori_loop(0, NCHUNK, fire, 0)

        def drain(j, carry):
            pltpu.make_async_copy(ones_v, acc_sh.at[didx.at[j]], sem_s).wait()
            return carry

        lax.fori_loop(0, NCHUNK, drain, 0)
        plsc.subcore_barrier()
        pltpu.sync_copy(acc_sh.at[pl.ds(r0, RPW)], out_hbm.at[c, pl.ds(r0, RPW)])

    HALF = NCHUNK // 2

    @functools.partial(
        pl.kernel,
        mesh=mesh,
        compiler_params=params_nl,
        out_type=jax.ShapeDtypeStruct((NC, N_PAD, H), jnp.float32),
        scratch_types=[
            pltpu.VMEM_SHARED((N_PAD, H), jnp.float32),
            pltpu.VMEM_SHARED((N_PAD, H), jnp.float32),
            pltpu.VMEM((NCHUNK, CHUNK), jnp.int32),
            pltpu.VMEM((NCHUNK, CHUNK), jnp.int32),
            pltpu.VMEM((HALF, CHUNK, H), jnp.float32),
            pltpu.VMEM((RPW, H), jnp.float32),
            pltpu.VMEM((RPW, H), jnp.float32),
            pltpu.VMEM((RPW, H), jnp.float32),
            pltpu.SemaphoreType.DMA,
            pltpu.SemaphoreType.DMA,
        ],
    )
    def _sc_scale_prop(xw_hbm, degp_hbm, src_hbm, dst_hbm, out_hbm, y_sh,
                       acc_sh, sidx, didx, rows, xw_v, d0_v, d1_v,
                       sem_g, sem_s):
        c = lax.axis_index("c")
        s = lax.axis_index("s")
        w = c * NS + s
        r0 = s * RPW
        pltpu.sync_copy(xw_hbm.at[pl.ds(r0, RPW)], xw_v)
        pltpu.sync_copy(degp_hbm.at[0, pl.ds(r0, RPW)], d0_v)
        pltpu.sync_copy(degp_hbm.at[1, pl.ds(r0, RPW)], d1_v)
        pltpu.sync_copy(src_hbm.at[pl.ds(w * NCHUNK, NCHUNK)], sidx)
        pltpu.sync_copy(dst_hbm.at[pl.ds(w * NCHUNK, NCHUNK)], didx)

        def row(r, carry):
            d = d0_v[r, :] + d1_v[r, :] - 1.0
            seed = jnp.int32(0x5F3759DF) - lax.shift_right_arithmetic(
                plsc.bitcast(d, jnp.int32), 1)
            v = plsc.bitcast(seed, jnp.float32)
            for _ in range(3):
                v = v * (1.5 - 0.5 * d * v * v)
            xw_v[r, :] = xw_v[r, :] * v
            d0_v[r, :] = jnp.zeros((H,), jnp.float32)
            return carry

        lax.fori_loop(0, RPW, row, 0)
        pltpu.sync_copy(xw_v, y_sh.at[pl.ds(r0, RPW)])

        @pl.when(c == 0)
        def _():
            pltpu.sync_copy(xw_v, acc_sh.at[pl.ds(r0, RPW)])

        @pl.when(c != 0)
        def _():
            pltpu.sync_copy(d0_v, acc_sh.at[pl.ds(r0, RPW)])

        plsc.subcore_barrier()

        for half in range(2):
            base = half * HALF

            def fire_g(j, carry):
                pltpu.async_copy(y_sh.at[sidx.at[base + j]], rows.at[j],
                                 sem_g)
                return carry

            lax.fori_loop(0, HALF, fire_g, 0)

            def drain_g(j, carry):
                pltpu.make_async_copy(y_sh.at[sidx.at[base + j]],
                                      rows.at[j], sem_g).wait()
                return carry

            lax.fori_loop(0, HALF, drain_g, 0)

            def fire_s(j, carry):
                pltpu.async_copy(rows.at[j], acc_sh.at[didx.at[base + j]],
                                 sem_s, add=True)
                return carry

            lax.fori_loop(0, HALF, fire_s, 0)

            def drain_s(j, carry):
                pltpu.make_async_copy(rows.at[j],
                                      acc_sh.at[didx.at[base + j]],
                                      sem_s).wait()
                return carry

            lax.fori_loop(0, HALF, drain_s, 0)

        plsc.subcore_barrier()
        pltpu.sync_copy(acc_sh.at[pl.ds(r0, RPW)], out_hbm.at[c, pl.ds(r0, RPW)])

    @functools.partial(
        pl.kernel,
        mesh=mesh,
        compiler_params=params,
        out_type=jax.ShapeDtypeStruct((NC, N_PAD, H), jnp.float32),
        scratch_types=[
            pltpu.VMEM_SHARED((N_PAD, H), jnp.float32),
            pltpu.VMEM((NCHUNK, CHUNK), jnp.int32),
            pltpu.VMEM((NCHUNK, CHUNK), jnp.int32),
            pltpu.VMEM((NCHUNK, CHUNK, H), jnp.float32),
            pltpu.SemaphoreType.DMA,
            pltpu.SemaphoreType.DMA,
        ],
    )
    def _sc_propagate(y_hbm, src_hbm, dst_hbm, out_hbm, acc_sh, sidx,
                      didx, rows, sem_g, sem_s):
        c = lax.axis_index("c")
        s = lax.axis_index("s")
        w = c * NS + s
        r0 = s * RPW
        pltpu.sync_copy(y_hbm.at[pl.ds(r0, RPW)], acc_sh.at[pl.ds(r0, RPW)])
        pltpu.sync_copy(src_hbm.at[pl.ds(w * NCHUNK, NCHUNK)], sidx)
        pltpu.sync_copy(dst_hbm.at[pl.ds(w * NCHUNK, NCHUNK)], didx)
        plsc.subcore_barrier()

        def fire_g(j, carry):
            pltpu.async_copy(y_hbm.at[sidx.at[j]], rows.at[j], sem_g)
            return carry

        lax.fori_loop(0, NCHUNK, fire_g, 0)

        def drain_g(j, carry):
            pltpu.make_async_copy(y_hbm.at[sidx.at[j]], rows.at[j],
                                  sem_g).wait()
            return carry

        lax.fori_loop(0, NCHUNK, drain_g, 0)

        def fire_s(j, carry):
            pltpu.async_copy(rows.at[j], acc_sh.at[didx.at[j]], sem_s,
                             add=True)
            return carry

        lax.fori_loop(0, NCHUNK, fire_s, 0)

        def drain_s(j, carry):
            pltpu.make_async_copy(rows.at[j], acc_sh.at[didx.at[j]],
                                  sem_s).wait()
            return carry

        lax.fori_loop(0, NCHUNK, drain_s, 0)
        plsc.subcore_barrier()
        pltpu.sync_copy(acc_sh.at[pl.ds(r0, RPW)], out_hbm.at[c, pl.ds(r0, RPW)])

    return _sc_degree, _sc_scale_prop, _sc_propagate


def _pdinv(degp0, degp1):
    return lax.rsqrt(jnp.maximum(degp0 + degp1 - 1.0, 1.0))


def _mm_body(xt_ref, w_ref, xw_ref):
    i = pl.program_id(0)
    xw = lax.dot_general(xt_ref[...], w_ref[...], (((0,), (0,)), ((), ())),
                         preferred_element_type=jnp.float32)
    rid = i * BLK + lax.broadcasted_iota(jnp.int32, (BLK, H), 0)
    xw_ref[...] = jnp.where(rid < N, xw, 0.0)


def _mid_body(accp_ref, degp_ref, b1t_ref, yh_ref):
    i = pl.program_id(0)
    dinv = _pdinv(degp_ref[0], degp_ref[1])
    acc = accp_ref[0] + accp_ref[1]
    h = jnp.maximum(dinv * acc + b1t_ref[...], 0.0)
    pr = i * PBLK + lax.broadcasted_iota(jnp.int32, (PBLK, 128), 0)
    yh_ref[...] = jnp.where(pr < PMASK, dinv * h, 0.0)


def _t_body(accp_ref, degp_ref, yh_ref, t_ref):
    dinv = _pdinv(degp_ref[0], degp_ref[1])
    t_ref[...] = dinv * (accp_ref[0] + accp_ref[1] - yh_ref[...])


def _final_body(t_ref, w2t_ref, b2_ref, out_ref):
    z = t_ref[...]
    w2t = w2t_ref[...]
    b2 = b2_ref[...]
    ls = [
        jnp.sum(z * w2t[c_:c_ + 1, :], axis=1, keepdims=True) + b2[0, c_]
        for c_ in range(C)
    ]
    m = jnp.maximum(jnp.maximum(ls[0], ls[1]), ls[2])
    se = sum(jnp.exp(l - m) for l in ls)
    lse = m + jnp.log(se)
    out_ref[...] = jnp.concatenate([l - lse for l in ls], axis=1)


_pk_spec = pl.BlockSpec((PBLK, 128), lambda i: (i, 0))
_pk_shape = jax.ShapeDtypeStruct((PN, 128), jnp.float32)
_pk2_spec = pl.BlockSpec((NC, PBLK, 128), lambda i: (0, i, 0))

_mm = pl.pallas_call(
    _mm_body,
    grid=(GRID,),
    in_specs=[
        pl.BlockSpec((D_IN, BLK), lambda i: (0, i)),
        pl.BlockSpec((D_IN, H), lambda i: (0, 0)),
    ],
    out_specs=pl.BlockSpec((BLK, H), lambda i: (i, 0)),
    out_shape=jax.ShapeDtypeStruct((N_PAD, H), jnp.float32),
)

_mid = pl.pallas_call(
    _mid_body,
    grid=(GRID,),
    in_specs=[
        _pk2_spec,
        _pk2_spec,
        pl.BlockSpec((1, 128), lambda i: (0, 0)),
    ],
    out_specs=_pk_spec,
    out_shape=_pk_shape,
)

_t = pl.pallas_call(
    _t_body,
    grid=(GRID,),
    in_specs=[_pk2_spec, _pk2_spec, _pk_spec],
    out_specs=_pk_spec,
    out_shape=_pk_shape,
)

_final = pl.pallas_call(
    _final_body,
    grid=(GRID,),
    in_specs=[
        pl.BlockSpec((BLK, H), lambda i: (i, 0)),
        pl.BlockSpec((C, H), lambda i: (0, 0)),
        pl.BlockSpec((1, C), lambda i: (0, 0)),
    ],
    out_specs=pl.BlockSpec((BLK, C), lambda i: (i, 0)),
    out_shape=jax.ShapeDtypeStruct((N_PAD, C), jnp.float32),
)


def kernel(x, edge_index, W1, b1, W2, b2):
    src = edge_index[0]
    dst = edge_index[1]
    npad = E_PAD - E
    pad_ids = (N + (jnp.arange(npad, dtype=jnp.int32) % (N_PAD - N)))
    pad_ids = pad_ids.astype(jnp.int32)
    src_p = jnp.concatenate([src, pad_ids]).reshape(E_PAD // CHUNK, CHUNK)
    dst_p = jnp.concatenate([dst, pad_ids]).reshape(E_PAD // CHUNK, CHUNK)

    ones = jnp.ones((N_PAD, H), jnp.float32)
    b1t = jnp.tile(b1, 128 // H).reshape(1, 128)

    sc_degree, sc_scale_prop, sc_propagate = _sc_kernels()
    degp = sc_degree(ones, dst_p)
    xw = _mm(x.T, W1)
    acc1 = sc_scale_prop(xw, degp, src_p, dst_p)
    degpp = degp.reshape(NC, PN, 128)
    yhp = _mid(acc1.reshape(NC, PN, 128), degpp, b1t)
    acc2 = sc_propagate(yhp.reshape(N_PAD, H), src_p, dst_p)
    tp = _t(acc2.reshape(NC, PN, 128), degpp, yhp)
    out = _final(tp.reshape(N_PAD, H), W2.T, b2.reshape(1, C))
    return out[:N]
